# Initial kernel scaffold; baseline (speedup 1.0000x reference)
import jax
import jax.numpy as jnp
from jax.experimental import pallas as pl


def kernel(feature, pos, pos_up, W, b):
    raise NotImplementedError("write your pallas kernel here")


# speedup vs baseline: 27.9011x; 27.9011x over previous
import functools

import jax
import jax.numpy as jnp
from jax.experimental import pallas as pl

_B, _N, _N_UP, _C, _K, _DIM = 2, 4096, 8192, 128, 3, 128
_TILE_Q = 256


def _up_sample_body(q_ref, kT_ref, f_ref, w_ref, b_ref, o_ref):
    q = q_ref[0]
    kT = kT_ref[0]
    qx, qy, qz = q[:, 0:1], q[:, 1:2], q[:, 2:3]
    kx, ky, kz = kT[0:1, :], kT[1:2, :], kT[2:3, :]

    qsq = qx * qx + qy * qy + qz * qz
    ksq = kx * kx + ky * ky + kz * kz
    qk = jax.lax.dot(q.astype(jnp.bfloat16), kT.astype(jnp.bfloat16),
                     preferred_element_type=jnp.float32)
    d2 = (qsq + ksq) - 2.0 * qk

    iota = jax.lax.broadcasted_iota(jnp.int32, d2.shape, 1)
    d2m = d2
    sels, ws = [], []
    for _ in range(_K):
        m = jnp.min(d2m, axis=1, keepdims=True)
        amin = jnp.min(jnp.where(d2m == m, iota, _N), axis=1, keepdims=True)
        sel = iota == amin
        dist = jnp.sqrt(jnp.maximum(m, 1e-12))
        ws.append(1.0 / (dist + 1e-6))
        sels.append(sel)
        d2m = jnp.where(sel, jnp.float32(3e38), d2m)

    wsum = ws[0] + ws[1] + ws[2]
    S = (jnp.where(sels[0], ws[0] / wsum, 0.0)
         + jnp.where(sels[1], ws[1] / wsum, 0.0)
         + jnp.where(sels[2], ws[2] / wsum, 0.0))

    G = jax.lax.dot(S, f_ref[0], precision=jax.lax.Precision.HIGHEST,
                    preferred_element_type=jnp.float32)
    out = jax.lax.dot(G.astype(jnp.bfloat16), w_ref[...].astype(jnp.bfloat16),
                      preferred_element_type=jnp.float32)
    o_ref[0] = jnp.maximum(out + b_ref[...], 0.0)


@functools.partial(jax.jit, static_argnames=())
def kernel(feature, pos, pos_up, W, b):
    posT = jnp.swapaxes(pos, 1, 2)
    b2 = b.reshape(1, _DIM)
    grid = (_B, _N_UP // _TILE_Q)
    out = pl.pallas_call(
        _up_sample_body,
        grid=grid,
        in_specs=[
            pl.BlockSpec((1, _TILE_Q, 3), lambda bi, qi: (bi, qi, 0)),
            pl.BlockSpec((1, 3, _N), lambda bi, qi: (bi, 0, 0)),
            pl.BlockSpec((1, _N, _C), lambda bi, qi: (bi, 0, 0)),
            pl.BlockSpec((_C, _DIM), lambda bi, qi: (0, 0)),
            pl.BlockSpec((1, _DIM), lambda bi, qi: (0, 0)),
        ],
        out_specs=pl.BlockSpec((1, _TILE_Q, _DIM), lambda bi, qi: (bi, qi, 0)),
        out_shape=jax.ShapeDtypeStruct((_B, _N_UP, _DIM), jnp.float32),
    )(pos_up, posT, feature, W, b2)
    return out


# speedup vs baseline: 46.0686x; 1.6511x over previous
import functools

import jax
import jax.numpy as jnp
from jax.experimental import pallas as pl
from jax.experimental.pallas import tpu as pltpu
from jax.experimental.pallas import tpu_sc as plsc

_B, _N, _N_UP, _C, _K, _DIM = 2, 4096, 8192, 128, 3, 128
_TILE_Q = 256
_GW = 128


def _knn_body(q_ref, kT_ref, oi_ref, ow_ref):
    bi = pl.program_id(0)
    q = q_ref[0]
    kT = kT_ref[0]
    qx, qy, qz = q[:, 0:1], q[:, 1:2], q[:, 2:3]
    kx, ky, kz = kT[0:1, :], kT[1:2, :], kT[2:3, :]

    qsq = qx * qx + qy * qy + qz * qz
    ksq = kx * kx + ky * ky + kz * kz
    qk = jax.lax.dot(q.astype(jnp.bfloat16), kT.astype(jnp.bfloat16),
                     preferred_element_type=jnp.float32)
    d2 = (qsq + ksq) - 2.0 * qk

    iota = jax.lax.broadcasted_iota(jnp.int32, d2.shape, 1)
    d2m = d2
    idxs, ws = [], []
    for _ in range(_K):
        m = jnp.min(d2m, axis=1, keepdims=True)
        amin = jnp.min(jnp.where(d2m == m, iota, _N), axis=1, keepdims=True)
        dist = jnp.sqrt(jnp.maximum(m, 1e-12))
        ws.append(1.0 / (dist + 1e-6))
        idxs.append(amin)
        d2m = jnp.where(iota == amin, jnp.float32(3e38), d2m)

    wsum = ws[0] + ws[1] + ws[2]
    ow_ref[0] = jnp.concatenate([w / wsum for w in ws], axis=1)
    oi_ref[0] = jnp.concatenate(idxs, axis=1) + bi * _N


def _tc_knn(pos_up, posT):
    grid = (_B, _N_UP // _TILE_Q)
    return pl.pallas_call(
        _knn_body,
        grid=grid,
        in_specs=[
            pl.BlockSpec((1, _TILE_Q, 3), lambda bi, qi: (bi, qi, 0)),
            pl.BlockSpec((1, 3, _N), lambda bi, qi: (bi, 0, 0)),
        ],
        out_specs=[
            pl.BlockSpec((1, _TILE_Q, _K), lambda bi, qi: (bi, qi, 0)),
            pl.BlockSpec((1, _TILE_Q, _K), lambda bi, qi: (bi, qi, 0)),
        ],
        out_shape=[
            jax.ShapeDtypeStruct((_B, _N_UP, _K), jnp.int32),
            jax.ShapeDtypeStruct((_B, _N_UP, _K), jnp.float32),
        ],
    )(pos_up, posT)


def _sc_gather(feature2, flat_idx):
    num_idx = flat_idx.shape[1]
    mesh = plsc.VectorSubcoreMesh(core_axis_name="core",
                                  subcore_axis_name="subcore")

    @pl.kernel(out_type=jax.ShapeDtypeStruct((num_idx, _C), jnp.float32),
               mesh=mesh)
    def kern(x_hbm, i_hbm, o_hbm):
        def body(i_vmem, o_vmem):
            pltpu.sync_copy(x_hbm.at[i_vmem.at[0]], o_vmem)

        pltpu.emit_pipeline(
            body,
            grid=(num_idx // _GW,),
            in_specs=[pl.BlockSpec((1, _GW), index_map=lambda i: (0, i))],
            out_specs=[pl.BlockSpec((_GW, _C), index_map=lambda i: (i, 0))],
            core_axis_name=("core", "subcore"),
            dimension_semantics=(pltpu.PARALLEL,),
        )(i_hbm, o_hbm)

    return kern(feature2, flat_idx)


def _interp_body(g_ref, w_ref, wm_ref, b_ref, o_ref):
    wn = w_ref[0]
    G = (wn[:, 0:1] * g_ref[0, 0]
         + wn[:, 1:2] * g_ref[1, 0]
         + wn[:, 2:3] * g_ref[2, 0])
    out = jax.lax.dot(G.astype(jnp.bfloat16), wm_ref[...].astype(jnp.bfloat16),
                      preferred_element_type=jnp.float32)
    o_ref[0] = jnp.maximum(out + b_ref[...], 0.0)


def _tc_interp(gathered, weights, W, b2):
    grid = (_B, _N_UP // _TILE_Q)
    return pl.pallas_call(
        _interp_body,
        grid=grid,
        in_specs=[
            pl.BlockSpec((_K, 1, _TILE_Q, _C), lambda bi, qi: (0, bi, qi, 0)),
            pl.BlockSpec((1, _TILE_Q, _K), lambda bi, qi: (bi, qi, 0)),
            pl.BlockSpec((_C, _DIM), lambda bi, qi: (0, 0)),
            pl.BlockSpec((1, _DIM), lambda bi, qi: (0, 0)),
        ],
        out_specs=pl.BlockSpec((1, _TILE_Q, _DIM), lambda bi, qi: (bi, qi, 0)),
        out_shape=jax.ShapeDtypeStruct((_B, _N_UP, _DIM), jnp.float32),
    )(gathered, weights, W, b2)


@jax.jit
def kernel(feature, pos, pos_up, W, b):
    posT = jnp.swapaxes(pos, 1, 2)
    b2 = b.reshape(1, _DIM)
    idx, weights = _tc_knn(pos_up, posT)
    flat_idx = jnp.transpose(idx, (2, 0, 1)).reshape(1, _K * _B * _N_UP)
    feature2 = feature.reshape(_B * _N, _C)
    gathered = _sc_gather(feature2, flat_idx)
    gathered = gathered.reshape(_K, _B, _N_UP, _C)
    return _tc_interp(gathered, weights, W, b2)
